# baseline (speedup 1.0000x reference)
import jax
import jax.numpy as jnp
from jax.experimental import pallas as pl

D_MODEL = 768
N_VOICES = 4


def _pe_add_kernel(ts_ref, vw_ref, x_ref, o_ref):
    ts = ts_ref[...]
    vw = vw_ref[...]
    pe = jnp.concatenate([ts, ts, ts, ts], axis=1) + vw
    o_ref[...] = x_ref[...] + pe[None]


def kernel(x, timestep_emb, voice_emb):
    B, L, D = x.shape
    T = L // N_VOICES
    W = N_VOICES * D
    xw = x.reshape(B, T, W)
    vw = voice_emb.reshape(1, W)
    ts = timestep_emb[:T]

    BT = 256
    grid = (T // BT, B)
    out = pl.pallas_call(
        _pe_add_kernel,
        grid=grid,
        in_specs=[
            pl.BlockSpec((BT, D), lambda i, b: (i, 0)),
            pl.BlockSpec((1, W), lambda i, b: (0, 0)),
            pl.BlockSpec((1, BT, W), lambda i, b: (b, i, 0)),
        ],
        out_specs=pl.BlockSpec((1, BT, W), lambda i, b: (b, i, 0)),
        out_shape=jax.ShapeDtypeStruct((B, T, W), x.dtype),
    )(ts, vw, xw)
    return out.reshape(B, L, D)
